# baseline (speedup 1.0000x reference)
import functools

import jax
import jax.numpy as jnp
from jax import lax
from jax.experimental import pallas as pl
from jax.experimental.pallas import tpu as pltpu
from jax.experimental.pallas import tpu_sc as plsc

_NC = 2
_NS = 16
_NW = _NC * _NS
_LANES = 16
_CHUNK = 32


def _prep_kernel(embed_ref, wout_ref, bout_ref, p_ref, e2_ref):
    E = embed_ref[0]
    heads = pl.num_programs(0)
    p_ref[...] = (
        jnp.dot(E, wout_ref[...], preferred_element_type=jnp.float32)
        + (1.0 / heads) * bout_ref[...][None, :]
    )
    e2_ref[...] = (0.5 * jnp.sum(E * E, axis=1))[None, None, :]


def _main_kernel(x_ref, win_ref, bin_ref, embed_ref, e2_ref,
                 fidx_ref, loss_ref):
    i = pl.program_id(0)
    heads, k, d = embed_ref.shape
    bn = x_ref.shape[0]
    wpb, n_chunks, heads_c = fidx_ref.shape
    c = heads_c // heads
    xi = (
        jnp.dot(x_ref[...], win_ref[...], preferred_element_type=jnp.float32)
        + bin_ref[...][None, :]
    )
    max_sum = jnp.float32(0.0)
    iota_f = lax.broadcasted_iota(jnp.int32, (bn, k), 1).astype(jnp.float32)
    idx_parts = []
    for h in range(heads):
        xi_h = xi[:, h * d:(h + 1) * d]
        dots = lax.dot_general(
            xi_h, embed_ref[h], (((1,), (1,)), ((), ())),
            preferred_element_type=jnp.float32)
        score = dots - e2_ref[h]
        maxv = jnp.max(score, axis=1, keepdims=True)
        idx = jnp.min(jnp.where(score == maxv, iota_f, jnp.float32(k)),
                      axis=1).astype(jnp.int32)
        idx_parts.append((idx + h * k).reshape(wpb * n_chunks, c))
        max_sum = max_sum + jnp.sum(maxv)
    fidx_ref[...] = jnp.concatenate(
        idx_parts, axis=1).reshape(wpb, n_chunks, heads * c)
    part = jnp.sum(xi * xi) - 2.0 * max_sum
    prev = jnp.where(i == 0, jnp.zeros((1, 1), jnp.float32), loss_ref[...])
    loss_ref[...] = prev + part


def _sc_gather_body(p_hbm, fidx_hbm, out_hbm, idx_v, buf_v, acc_v,
                    gsem0, gsem1, osem0, osem1):
    n_chunks = fidx_hbm.shape[1]
    heads_c = fidx_hbm.shape[2]
    dim = p_hbm.shape[1]
    c = _CHUNK
    heads = heads_c // c
    rows_per_w = n_chunks * c
    gsems = (gsem0, gsem1)
    osems = (osem0, osem1)
    wid = lax.axis_index("s") * _NC + lax.axis_index("c")
    base = wid * rows_per_w
    pltpu.sync_copy(fidx_hbm.at[wid], idx_v)

    def fire(ci):
        p = ci % 2
        return pltpu.async_copy(
            p_hbm.at[idx_v.at[ci]], buf_v.at[p], gsems[p])

    gathers = {0: fire(0)}
    out_copies = {}
    for ci in range(n_chunks):
        p = ci % 2
        if ci + 1 < n_chunks:
            gathers[ci + 1] = fire(ci + 1)
        gathers.pop(ci).wait()
        if ci >= 2:
            out_copies.pop(ci - 2).wait()

        def body(r, _):
            for v in range(dim // _LANES):
                s = pl.ds(v * _LANES, _LANES)
                acc_v[p, r, s] = (
                    (buf_v[p, 0 * c + r, s] + buf_v[p, 1 * c + r, s])
                    + (buf_v[p, 2 * c + r, s] + buf_v[p, 3 * c + r, s]))
            return 0

        lax.fori_loop(0, c, body, 0)
        out_copies[ci] = pltpu.async_copy(
            acc_v.at[p], out_hbm.at[pl.ds(base + ci * c, c)], osems[p])
    for ci in sorted(out_copies):
        out_copies.pop(ci).wait()


_NSLICE = 8


def kernel(x, W_in, b_in, W_out, b_out, embed):
    n, dim = x.shape
    heads, k, d = embed.shape
    in_dim = heads * d
    ns = n // _NSLICE
    rows_per_w = ns // _NW
    bn = 512
    wpb = bn // rows_per_w
    grid = ns // bn
    n_chunks = rows_per_w // _CHUNK

    P, e2 = pl.pallas_call(
        _prep_kernel,
        grid=(heads,),
        in_specs=[
            pl.BlockSpec((1, k, d), lambda h: (h, 0, 0)),
            pl.BlockSpec((d, dim), lambda h: (h, 0)),
            pl.BlockSpec((dim,), lambda h: (0,)),
        ],
        out_specs=[
            pl.BlockSpec((k, dim), lambda h: (h, 0)),
            pl.BlockSpec((1, 1, k), lambda h: (h, 0, 0)),
        ],
        out_shape=[
            jax.ShapeDtypeStruct((heads * k, dim), jnp.float32),
            jax.ShapeDtypeStruct((heads, 1, k), jnp.float32),
        ],
    )(embed, W_out, b_out)

    main_call = pl.pallas_call(
        _main_kernel,
        grid=(grid,),
        in_specs=[
            pl.BlockSpec((bn, dim), lambda i: (i, 0)),
            pl.BlockSpec((dim, in_dim), lambda i: (0, 0)),
            pl.BlockSpec((in_dim,), lambda i: (0,)),
            pl.BlockSpec((heads, k, d), lambda i: (0, 0, 0)),
            pl.BlockSpec((heads, 1, k), lambda i: (0, 0, 0)),
        ],
        out_specs=[
            pl.BlockSpec((wpb, n_chunks, heads * _CHUNK), lambda i: (i, 0, 0)),
            pl.BlockSpec((1, 1), lambda i: (0, 0)),
        ],
        out_shape=[
            jax.ShapeDtypeStruct((_NW, n_chunks, heads * _CHUNK), jnp.int32),
            jax.ShapeDtypeStruct((1, 1), jnp.float32),
        ],
    )

    sc_gather = functools.partial(
        pl.kernel,
        out_type=jax.ShapeDtypeStruct((ns, dim), jnp.float32),
        mesh=plsc.VectorSubcoreMesh(core_axis_name="c", subcore_axis_name="s"),
        scratch_types=[
            pltpu.VMEM((n_chunks, heads * _CHUNK), jnp.int32),
            pltpu.VMEM((2, heads * _CHUNK, dim), jnp.float32),
            pltpu.VMEM((2, _CHUNK, dim), jnp.float32),
            pltpu.SemaphoreType.DMA,
            pltpu.SemaphoreType.DMA,
            pltpu.SemaphoreType.DMA,
            pltpu.SemaphoreType.DMA,
        ],
    )(_sc_gather_body)

    outs = []
    loss = jnp.zeros((), jnp.float32)
    for s in range(_NSLICE):
        xs = lax.slice_in_dim(x, s * ns, (s + 1) * ns, axis=0)
        fidx_s, loss_s = main_call(xs, W_in, b_in, embed, e2)
        outs.append(sc_gather(P, fidx_s))
        loss = loss + loss_s[0, 0]

    out = jnp.concatenate(outs, axis=0)
    l_vq = loss / jnp.float32(heads * n * d)
    return (out, l_vq)


# speedup vs baseline: 1.3393x; 1.3393x over previous
import functools

import jax
import jax.numpy as jnp
from jax import lax
from jax.experimental import pallas as pl
from jax.experimental.pallas import tpu as pltpu
from jax.experimental.pallas import tpu_sc as plsc

_NC = 2
_NS = 16
_NW = _NC * _NS
_LANES = 16
_CHUNK = 32


def _prep_kernel(embed_ref, wout_ref, bout_ref, p_ref, e2_ref):
    E = embed_ref[0]
    heads = pl.num_programs(0)
    p_ref[...] = (
        jnp.dot(E, wout_ref[...], preferred_element_type=jnp.float32)
        + (1.0 / heads) * bout_ref[...][None, :]
    )
    e2_ref[...] = (0.5 * jnp.sum(E * E, axis=1))[None, None, :]


def _main_kernel(x_ref, win_ref, bin_ref, embed_ref, e2_ref,
                 fidx_ref, loss_ref):
    i = pl.program_id(0)
    heads, k, d = embed_ref.shape
    bn = x_ref.shape[0]
    wpb, n_chunks, heads_c = fidx_ref.shape
    c = heads_c // heads
    xi = (
        jnp.dot(x_ref[...], win_ref[...], preferred_element_type=jnp.float32)
        + bin_ref[...][None, :]
    )
    max_sum = jnp.float32(0.0)
    iota_f = lax.broadcasted_iota(jnp.int32, (bn, k), 1).astype(jnp.float32)
    idx_parts = []
    for h in range(heads):
        xi_h = xi[:, h * d:(h + 1) * d]
        dots = lax.dot_general(
            xi_h, embed_ref[h], (((1,), (1,)), ((), ())),
            preferred_element_type=jnp.float32)
        score = dots - e2_ref[h]
        maxv = jnp.max(score, axis=1, keepdims=True)
        idx = jnp.min(jnp.where(score == maxv, iota_f, jnp.float32(k)),
                      axis=1).astype(jnp.int32)
        idx_parts.append((idx + h * k).reshape(wpb * n_chunks, c))
        max_sum = max_sum + jnp.sum(maxv)
    fidx_ref[...] = jnp.concatenate(
        idx_parts, axis=1).reshape(wpb, n_chunks, heads * c)
    part = jnp.sum(xi * xi) - 2.0 * max_sum
    prev = jnp.where(i == 0, jnp.zeros((1, 1), jnp.float32), loss_ref[...])
    loss_ref[...] = prev + part


def _main_out_kernel(x_ref, win_ref, bin_ref, embed_ref, e2_ref, p_ref,
                     out_ref, loss_ref):
    i = pl.program_id(0)
    heads, k, d = embed_ref.shape
    bn = x_ref.shape[0]
    xi = (
        jnp.dot(x_ref[...], win_ref[...], preferred_element_type=jnp.float32)
        + bin_ref[...][None, :]
    )
    max_sum = jnp.float32(0.0)
    iota_f = lax.broadcasted_iota(jnp.int32, (bn, k), 1).astype(jnp.float32)
    out = jnp.zeros(out_ref.shape, jnp.float32)
    for h in range(heads):
        xi_h = xi[:, h * d:(h + 1) * d]
        dots = lax.dot_general(
            xi_h, embed_ref[h], (((1,), (1,)), ((), ())),
            preferred_element_type=jnp.float32)
        score = dots - e2_ref[h]
        maxv = jnp.max(score, axis=1, keepdims=True)
        idxf = jnp.min(jnp.where(score == maxv, iota_f, jnp.float32(k)),
                       axis=1)
        onehot = (iota_f == idxf[:, None]).astype(jnp.float32)
        out = out + jnp.dot(onehot, p_ref[pl.ds(h * k, k), :],
                            preferred_element_type=jnp.float32)
        max_sum = max_sum + jnp.sum(maxv)
    out_ref[...] = out
    part = jnp.sum(xi * xi) - 2.0 * max_sum
    prev = jnp.where(i == 0, jnp.zeros((1, 1), jnp.float32), loss_ref[...])
    loss_ref[...] = prev + part


def _sc_gather_body(p_hbm, fidx_hbm, out_hbm, idx_v, buf_v, acc_v,
                    gsem0, gsem1, osem0, osem1):
    n_chunks = fidx_hbm.shape[1]
    heads_c = fidx_hbm.shape[2]
    dim = p_hbm.shape[1]
    c = _CHUNK
    heads = heads_c // c
    rows_per_w = n_chunks * c
    gsems = (gsem0, gsem1)
    osems = (osem0, osem1)
    wid = lax.axis_index("s") * _NC + lax.axis_index("c")
    base = wid * rows_per_w
    pltpu.sync_copy(fidx_hbm.at[wid], idx_v)

    def fire(ci):
        p = ci % 2
        return pltpu.async_copy(
            p_hbm.at[idx_v.at[ci]], buf_v.at[p], gsems[p])

    gathers = {0: fire(0)}
    out_copies = {}
    for ci in range(n_chunks):
        p = ci % 2
        if ci + 1 < n_chunks:
            gathers[ci + 1] = fire(ci + 1)
        gathers.pop(ci).wait()
        if ci >= 2:
            out_copies.pop(ci - 2).wait()

        def body(r, _):
            for v in range(dim // _LANES):
                s = pl.ds(v * _LANES, _LANES)
                acc_v[p, r, s] = (
                    (buf_v[p, 0 * c + r, s] + buf_v[p, 1 * c + r, s])
                    + (buf_v[p, 2 * c + r, s] + buf_v[p, 3 * c + r, s]))
            return 0

        lax.fori_loop(0, c, body, 0)
        out_copies[ci] = pltpu.async_copy(
            acc_v.at[p], out_hbm.at[pl.ds(base + ci * c, c)], osems[p])
    for ci in sorted(out_copies):
        out_copies.pop(ci).wait()


_NSLICE = 4


def kernel(x, W_in, b_in, W_out, b_out, embed):
    n, dim = x.shape
    heads, k, d = embed.shape
    in_dim = heads * d
    ns = n // _NSLICE
    rows_per_w = ns // _NW
    bn = 512
    wpb = bn // rows_per_w
    grid = ns // bn
    n_chunks = rows_per_w // _CHUNK

    P, e2 = pl.pallas_call(
        _prep_kernel,
        grid=(heads,),
        in_specs=[
            pl.BlockSpec((1, k, d), lambda h: (h, 0, 0)),
            pl.BlockSpec((d, dim), lambda h: (h, 0)),
            pl.BlockSpec((dim,), lambda h: (0,)),
        ],
        out_specs=[
            pl.BlockSpec((k, dim), lambda h: (h, 0)),
            pl.BlockSpec((1, 1, k), lambda h: (h, 0, 0)),
        ],
        out_shape=[
            jax.ShapeDtypeStruct((heads * k, dim), jnp.float32),
            jax.ShapeDtypeStruct((heads, 1, k), jnp.float32),
        ],
    )(embed, W_out, b_out)

    main_call = pl.pallas_call(
        _main_kernel,
        grid=(grid,),
        in_specs=[
            pl.BlockSpec((bn, dim), lambda i: (i, 0)),
            pl.BlockSpec((dim, in_dim), lambda i: (0, 0)),
            pl.BlockSpec((in_dim,), lambda i: (0,)),
            pl.BlockSpec((heads, k, d), lambda i: (0, 0, 0)),
            pl.BlockSpec((heads, 1, k), lambda i: (0, 0, 0)),
        ],
        out_specs=[
            pl.BlockSpec((wpb, n_chunks, heads * _CHUNK), lambda i: (i, 0, 0)),
            pl.BlockSpec((1, 1), lambda i: (0, 0)),
        ],
        out_shape=[
            jax.ShapeDtypeStruct((_NW, n_chunks, heads * _CHUNK), jnp.int32),
            jax.ShapeDtypeStruct((1, 1), jnp.float32),
        ],
    )

    sc_gather = functools.partial(
        pl.kernel,
        out_type=jax.ShapeDtypeStruct((ns, dim), jnp.float32),
        mesh=plsc.VectorSubcoreMesh(core_axis_name="c", subcore_axis_name="s"),
        scratch_types=[
            pltpu.VMEM((n_chunks, heads * _CHUNK), jnp.int32),
            pltpu.VMEM((2, heads * _CHUNK, dim), jnp.float32),
            pltpu.VMEM((2, _CHUNK, dim), jnp.float32),
            pltpu.SemaphoreType.DMA,
            pltpu.SemaphoreType.DMA,
            pltpu.SemaphoreType.DMA,
            pltpu.SemaphoreType.DMA,
        ],
    )(_sc_gather_body)

    out_call = pl.pallas_call(
        _main_out_kernel,
        grid=(grid,),
        in_specs=[
            pl.BlockSpec((bn, dim), lambda i: (i, 0)),
            pl.BlockSpec((dim, in_dim), lambda i: (0, 0)),
            pl.BlockSpec((in_dim,), lambda i: (0,)),
            pl.BlockSpec((heads, k, d), lambda i: (0, 0, 0)),
            pl.BlockSpec((heads, 1, k), lambda i: (0, 0, 0)),
            pl.BlockSpec((heads * k, dim), lambda i: (0, 0)),
        ],
        out_specs=[
            pl.BlockSpec((bn, dim), lambda i: (i, 0)),
            pl.BlockSpec((1, 1), lambda i: (0, 0)),
        ],
        out_shape=[
            jax.ShapeDtypeStruct((ns, dim), jnp.float32),
            jax.ShapeDtypeStruct((1, 1), jnp.float32),
        ],
    )

    outs = []
    loss = jnp.zeros((), jnp.float32)
    for s in range(_NSLICE):
        xs = lax.slice_in_dim(x, s * ns, (s + 1) * ns, axis=0)
        if s == _NSLICE - 1:
            out_s, loss_s = out_call(xs, W_in, b_in, embed, e2, P)
            outs.append(out_s)
            loss = loss + loss_s[0, 0]
            continue
        fidx_s, loss_s = main_call(xs, W_in, b_in, embed, e2)
        outs.append(sc_gather(P, fidx_s))
        loss = loss + loss_s[0, 0]

    out = jnp.concatenate(outs, axis=0)
    l_vq = loss / jnp.float32(heads * n * d)
    return (out, l_vq)


# speedup vs baseline: 1.3537x; 1.0108x over previous
import functools

import jax
import jax.numpy as jnp
from jax import lax
from jax.experimental import pallas as pl
from jax.experimental.pallas import tpu as pltpu
from jax.experimental.pallas import tpu_sc as plsc

_NC = 2
_NS = 16
_NW = _NC * _NS
_LANES = 16
_CHUNK = 32


def _prep_kernel(embed_ref, wout_ref, bout_ref, p_ref, e2_ref):
    E = embed_ref[0]
    heads = pl.num_programs(0)
    p_ref[...] = (
        jnp.dot(E, wout_ref[...], preferred_element_type=jnp.float32)
        + (1.0 / heads) * bout_ref[...][None, :]
    )
    e2_ref[...] = (0.5 * jnp.sum(E * E, axis=1))[None, None, :]


def _main_kernel(x_ref, win_ref, bin_ref, embed_ref, e2_ref,
                 fidx_ref, loss_ref):
    i = pl.program_id(0)
    heads, k, d = embed_ref.shape
    bn = x_ref.shape[0]
    wpb, n_chunks, heads_c = fidx_ref.shape
    c = heads_c // heads
    xi = (
        jnp.dot(x_ref[...], win_ref[...], preferred_element_type=jnp.float32)
        + bin_ref[...][None, :]
    )
    max_sum = jnp.float32(0.0)
    iota_f = lax.broadcasted_iota(jnp.int32, (bn, k), 1).astype(jnp.float32)
    idx_parts = []
    for h in range(heads):
        xi_h = xi[:, h * d:(h + 1) * d]
        dots = lax.dot_general(
            xi_h, embed_ref[h], (((1,), (1,)), ((), ())),
            preferred_element_type=jnp.float32)
        score = dots - e2_ref[h]
        maxv = jnp.max(score, axis=1, keepdims=True)
        idx = jnp.min(jnp.where(score == maxv, iota_f, jnp.float32(k)),
                      axis=1).astype(jnp.int32)
        idx_parts.append((idx + h * k).reshape(wpb * n_chunks, c))
        max_sum = max_sum + jnp.sum(maxv)
    fidx_ref[...] = jnp.concatenate(
        idx_parts, axis=1).reshape(wpb, n_chunks, heads * c)
    part = jnp.sum(xi * xi) - 2.0 * max_sum
    prev = jnp.where(i == 0, jnp.zeros((1, 1), jnp.float32), loss_ref[...])
    loss_ref[...] = prev + part


def _main_out_kernel(x_ref, win_ref, bin_ref, embed_ref, e2_ref, p_ref,
                     out_ref, loss_ref):
    i = pl.program_id(0)
    heads, k, d = embed_ref.shape
    bn = x_ref.shape[0]
    xi = (
        jnp.dot(x_ref[...], win_ref[...], preferred_element_type=jnp.float32)
        + bin_ref[...][None, :]
    )
    max_sum = jnp.float32(0.0)
    iota_f = lax.broadcasted_iota(jnp.int32, (bn, k), 1).astype(jnp.float32)
    out = jnp.zeros(out_ref.shape, jnp.float32)
    for h in range(heads):
        xi_h = xi[:, h * d:(h + 1) * d]
        dots = lax.dot_general(
            xi_h, embed_ref[h], (((1,), (1,)), ((), ())),
            preferred_element_type=jnp.float32)
        score = dots - e2_ref[h]
        maxv = jnp.max(score, axis=1, keepdims=True)
        idxf = jnp.min(jnp.where(score == maxv, iota_f, jnp.float32(k)),
                       axis=1)
        onehot = (iota_f == idxf[:, None]).astype(jnp.float32)
        out = out + jnp.dot(onehot, p_ref[pl.ds(h * k, k), :],
                            preferred_element_type=jnp.float32)
        max_sum = max_sum + jnp.sum(maxv)
    out_ref[...] = out
    part = jnp.sum(xi * xi) - 2.0 * max_sum
    prev = jnp.where(i == 0, jnp.zeros((1, 1), jnp.float32), loss_ref[...])
    loss_ref[...] = prev + part


def _sc_gather_body(p_hbm, fidx_hbm, out_hbm, idx_v, buf_v, acc_v,
                    gsem0, gsem1, osem0, osem1):
    n_chunks = fidx_hbm.shape[1]
    heads_c = fidx_hbm.shape[2]
    dim = p_hbm.shape[1]
    c = _CHUNK
    heads = heads_c // c
    rows_per_w = n_chunks * c
    gsems = (gsem0, gsem1)
    osems = (osem0, osem1)
    wid = lax.axis_index("s") * _NC + lax.axis_index("c")
    base = wid * rows_per_w
    pltpu.sync_copy(fidx_hbm.at[wid], idx_v)

    def fire(ci):
        p = ci % 2
        return pltpu.async_copy(
            p_hbm.at[idx_v.at[ci]], buf_v.at[p], gsems[p])

    gathers = {0: fire(0)}
    out_copies = {}
    for ci in range(n_chunks):
        p = ci % 2
        if ci + 1 < n_chunks:
            gathers[ci + 1] = fire(ci + 1)
        gathers.pop(ci).wait()
        if ci >= 2:
            out_copies.pop(ci - 2).wait()

        def body(r, _):
            for v in range(dim // _LANES):
                s = pl.ds(v * _LANES, _LANES)
                acc_v[p, r, s] = (
                    (buf_v[p, 0 * c + r, s] + buf_v[p, 1 * c + r, s])
                    + (buf_v[p, 2 * c + r, s] + buf_v[p, 3 * c + r, s]))
            return 0

        lax.fori_loop(0, c, body, 0)
        out_copies[ci] = pltpu.async_copy(
            acc_v.at[p], out_hbm.at[pl.ds(base + ci * c, c)], osems[p])
    for ci in sorted(out_copies):
        out_copies.pop(ci).wait()


_NSLICE = 4


def kernel(x, W_in, b_in, W_out, b_out, embed):
    n, dim = x.shape
    heads, k, d = embed.shape
    in_dim = heads * d
    ns = n // _NSLICE
    rows_per_w = ns // _NW
    bn = 1024
    wpb = bn // rows_per_w
    grid = ns // bn
    n_chunks = rows_per_w // _CHUNK

    P, e2 = pl.pallas_call(
        _prep_kernel,
        grid=(heads,),
        in_specs=[
            pl.BlockSpec((1, k, d), lambda h: (h, 0, 0)),
            pl.BlockSpec((d, dim), lambda h: (h, 0)),
            pl.BlockSpec((dim,), lambda h: (0,)),
        ],
        out_specs=[
            pl.BlockSpec((k, dim), lambda h: (h, 0)),
            pl.BlockSpec((1, 1, k), lambda h: (h, 0, 0)),
        ],
        out_shape=[
            jax.ShapeDtypeStruct((heads * k, dim), jnp.float32),
            jax.ShapeDtypeStruct((heads, 1, k), jnp.float32),
        ],
    )(embed, W_out, b_out)

    main_call = pl.pallas_call(
        _main_kernel,
        grid=(grid,),
        in_specs=[
            pl.BlockSpec((bn, dim), lambda i: (i, 0)),
            pl.BlockSpec((dim, in_dim), lambda i: (0, 0)),
            pl.BlockSpec((in_dim,), lambda i: (0,)),
            pl.BlockSpec((heads, k, d), lambda i: (0, 0, 0)),
            pl.BlockSpec((heads, 1, k), lambda i: (0, 0, 0)),
        ],
        out_specs=[
            pl.BlockSpec((wpb, n_chunks, heads * _CHUNK), lambda i: (i, 0, 0)),
            pl.BlockSpec((1, 1), lambda i: (0, 0)),
        ],
        out_shape=[
            jax.ShapeDtypeStruct((_NW, n_chunks, heads * _CHUNK), jnp.int32),
            jax.ShapeDtypeStruct((1, 1), jnp.float32),
        ],
    )

    sc_gather = functools.partial(
        pl.kernel,
        out_type=jax.ShapeDtypeStruct((ns, dim), jnp.float32),
        mesh=plsc.VectorSubcoreMesh(core_axis_name="c", subcore_axis_name="s"),
        scratch_types=[
            pltpu.VMEM((n_chunks, heads * _CHUNK), jnp.int32),
            pltpu.VMEM((2, heads * _CHUNK, dim), jnp.float32),
            pltpu.VMEM((2, _CHUNK, dim), jnp.float32),
            pltpu.SemaphoreType.DMA,
            pltpu.SemaphoreType.DMA,
            pltpu.SemaphoreType.DMA,
            pltpu.SemaphoreType.DMA,
        ],
    )(_sc_gather_body)

    out_call = pl.pallas_call(
        _main_out_kernel,
        grid=(grid,),
        in_specs=[
            pl.BlockSpec((bn, dim), lambda i: (i, 0)),
            pl.BlockSpec((dim, in_dim), lambda i: (0, 0)),
            pl.BlockSpec((in_dim,), lambda i: (0,)),
            pl.BlockSpec((heads, k, d), lambda i: (0, 0, 0)),
            pl.BlockSpec((heads, 1, k), lambda i: (0, 0, 0)),
            pl.BlockSpec((heads * k, dim), lambda i: (0, 0)),
        ],
        out_specs=[
            pl.BlockSpec((bn, dim), lambda i: (i, 0)),
            pl.BlockSpec((1, 1), lambda i: (0, 0)),
        ],
        out_shape=[
            jax.ShapeDtypeStruct((ns, dim), jnp.float32),
            jax.ShapeDtypeStruct((1, 1), jnp.float32),
        ],
    )

    outs = []
    loss = jnp.zeros((), jnp.float32)
    for s in range(_NSLICE):
        xs = lax.slice_in_dim(x, s * ns, (s + 1) * ns, axis=0)
        if s == _NSLICE - 1:
            out_s, loss_s = out_call(xs, W_in, b_in, embed, e2, P)
            outs.append(out_s)
            loss = loss + loss_s[0, 0]
            continue
        fidx_s, loss_s = main_call(xs, W_in, b_in, embed, e2)
        outs.append(sc_gather(P, fidx_s))
        loss = loss + loss_s[0, 0]

    out = jnp.concatenate(outs, axis=0)
    l_vq = loss / jnp.float32(heads * n * d)
    return (out, l_vq)


# speedup vs baseline: 1.3589x; 1.0038x over previous
import functools

import jax
import jax.numpy as jnp
from jax import lax
from jax.experimental import pallas as pl
from jax.experimental.pallas import tpu as pltpu
from jax.experimental.pallas import tpu_sc as plsc

_NC = 2
_NS = 16
_NW = _NC * _NS
_LANES = 16
_CHUNK = 32


def _prep_kernel(embed_ref, wout_ref, bout_ref, p_ref, e2_ref):
    E = embed_ref[0]
    heads = pl.num_programs(0)
    p_ref[...] = (
        jnp.dot(E, wout_ref[...], preferred_element_type=jnp.float32)
        + (1.0 / heads) * bout_ref[...][None, :]
    )
    e2_ref[...] = (0.5 * jnp.sum(E * E, axis=1))[None, None, :]


def _main_kernel(x_ref, win_ref, bin_ref, embed_ref, e2_ref,
                 fidx_ref, loss_ref):
    i = pl.program_id(0)
    heads, k, d = embed_ref.shape
    bn = x_ref.shape[0]
    wpb, n_chunks, heads_c = fidx_ref.shape
    c = heads_c // heads
    xi = (
        jnp.dot(x_ref[...], win_ref[...], preferred_element_type=jnp.float32)
        + bin_ref[...][None, :]
    )
    max_sum = jnp.float32(0.0)
    iota_f = lax.broadcasted_iota(jnp.int32, (bn, k), 1).astype(jnp.float32)
    idx_parts = []
    for h in range(heads):
        xi_h = xi[:, h * d:(h + 1) * d]
        dots = lax.dot_general(
            xi_h, embed_ref[h], (((1,), (1,)), ((), ())),
            preferred_element_type=jnp.float32)
        score = dots - e2_ref[h]
        maxv = jnp.max(score, axis=1, keepdims=True)
        idx = jnp.min(jnp.where(score == maxv, iota_f, jnp.float32(k)),
                      axis=1).astype(jnp.int32)
        idx_parts.append((idx + h * k).reshape(wpb * n_chunks, c))
        max_sum = max_sum + jnp.sum(maxv)
    fidx_ref[...] = jnp.concatenate(
        idx_parts, axis=1).reshape(wpb, n_chunks, heads * c)
    part = jnp.sum(xi * xi) - 2.0 * max_sum
    prev = jnp.where(i == 0, jnp.zeros((1, 1), jnp.float32), loss_ref[...])
    loss_ref[...] = prev + part


def _main_out_kernel(x_ref, win_ref, bin_ref, embed_ref, e2_ref, p_ref,
                     out_ref, loss_ref):
    i = pl.program_id(0)
    heads, k, d = embed_ref.shape
    bn = x_ref.shape[0]
    xi = (
        jnp.dot(x_ref[...], win_ref[...], preferred_element_type=jnp.float32)
        + bin_ref[...][None, :]
    )
    max_sum = jnp.float32(0.0)
    iota_f = lax.broadcasted_iota(jnp.int32, (bn, k), 1).astype(jnp.float32)
    out = jnp.zeros(out_ref.shape, jnp.float32)
    for h in range(heads):
        xi_h = xi[:, h * d:(h + 1) * d]
        dots = lax.dot_general(
            xi_h, embed_ref[h], (((1,), (1,)), ((), ())),
            preferred_element_type=jnp.float32)
        score = dots - e2_ref[h]
        maxv = jnp.max(score, axis=1, keepdims=True)
        idxf = jnp.min(jnp.where(score == maxv, iota_f, jnp.float32(k)),
                       axis=1)
        onehot = (iota_f == idxf[:, None]).astype(jnp.float32)
        out = out + jnp.dot(onehot, p_ref[pl.ds(h * k, k), :],
                            preferred_element_type=jnp.float32)
        max_sum = max_sum + jnp.sum(maxv)
    out_ref[...] = out
    part = jnp.sum(xi * xi) - 2.0 * max_sum
    prev = jnp.where(i == 0, jnp.zeros((1, 1), jnp.float32), loss_ref[...])
    loss_ref[...] = prev + part


def _sc_gather_body(p_hbm, fidx_hbm, out_hbm, idx_v, buf_v, acc_v,
                    gsem0, gsem1, osem0, osem1):
    n_chunks = fidx_hbm.shape[1]
    heads_c = fidx_hbm.shape[2]
    dim = p_hbm.shape[1]
    c = _CHUNK
    heads = heads_c // c
    rows_per_w = n_chunks * c
    gsems = (gsem0, gsem1)
    osems = (osem0, osem1)
    wid = lax.axis_index("s") * _NC + lax.axis_index("c")
    base = wid * rows_per_w
    pltpu.sync_copy(fidx_hbm.at[wid], idx_v)

    def fire(ci):
        p = ci % 2
        return pltpu.async_copy(
            p_hbm.at[idx_v.at[ci]], buf_v.at[p], gsems[p])

    gathers = {0: fire(0)}
    out_copies = {}
    for ci in range(n_chunks):
        p = ci % 2
        if ci + 1 < n_chunks:
            gathers[ci + 1] = fire(ci + 1)
        gathers.pop(ci).wait()
        if ci >= 2:
            out_copies.pop(ci - 2).wait()

        def body(r, _):
            for v in range(dim // _LANES):
                s = pl.ds(v * _LANES, _LANES)
                acc_v[p, r, s] = (
                    (buf_v[p, 0 * c + r, s] + buf_v[p, 1 * c + r, s])
                    + (buf_v[p, 2 * c + r, s] + buf_v[p, 3 * c + r, s]))
            return 0

        lax.fori_loop(0, c, body, 0)
        out_copies[ci] = pltpu.async_copy(
            acc_v.at[p], out_hbm.at[pl.ds(base + ci * c, c)], osems[p])
    for ci in sorted(out_copies):
        out_copies.pop(ci).wait()


_NSLICE = 4


def kernel(x, W_in, b_in, W_out, b_out, embed):
    n, dim = x.shape
    heads, k, d = embed.shape
    in_dim = heads * d
    ns = n // _NSLICE
    rows_per_w = ns // _NW
    bn = 2048
    wpb = bn // rows_per_w
    grid = ns // bn
    n_chunks = rows_per_w // _CHUNK

    P, e2 = pl.pallas_call(
        _prep_kernel,
        grid=(heads,),
        in_specs=[
            pl.BlockSpec((1, k, d), lambda h: (h, 0, 0)),
            pl.BlockSpec((d, dim), lambda h: (h, 0)),
            pl.BlockSpec((dim,), lambda h: (0,)),
        ],
        out_specs=[
            pl.BlockSpec((k, dim), lambda h: (h, 0)),
            pl.BlockSpec((1, 1, k), lambda h: (h, 0, 0)),
        ],
        out_shape=[
            jax.ShapeDtypeStruct((heads * k, dim), jnp.float32),
            jax.ShapeDtypeStruct((heads, 1, k), jnp.float32),
        ],
    )(embed, W_out, b_out)

    main_call = pl.pallas_call(
        _main_kernel,
        grid=(grid,),
        in_specs=[
            pl.BlockSpec((bn, dim), lambda i: (i, 0)),
            pl.BlockSpec((dim, in_dim), lambda i: (0, 0)),
            pl.BlockSpec((in_dim,), lambda i: (0,)),
            pl.BlockSpec((heads, k, d), lambda i: (0, 0, 0)),
            pl.BlockSpec((heads, 1, k), lambda i: (0, 0, 0)),
        ],
        out_specs=[
            pl.BlockSpec((wpb, n_chunks, heads * _CHUNK), lambda i: (i, 0, 0)),
            pl.BlockSpec((1, 1), lambda i: (0, 0)),
        ],
        out_shape=[
            jax.ShapeDtypeStruct((_NW, n_chunks, heads * _CHUNK), jnp.int32),
            jax.ShapeDtypeStruct((1, 1), jnp.float32),
        ],
    )

    sc_gather = functools.partial(
        pl.kernel,
        out_type=jax.ShapeDtypeStruct((ns, dim), jnp.float32),
        mesh=plsc.VectorSubcoreMesh(core_axis_name="c", subcore_axis_name="s"),
        scratch_types=[
            pltpu.VMEM((n_chunks, heads * _CHUNK), jnp.int32),
            pltpu.VMEM((2, heads * _CHUNK, dim), jnp.float32),
            pltpu.VMEM((2, _CHUNK, dim), jnp.float32),
            pltpu.SemaphoreType.DMA,
            pltpu.SemaphoreType.DMA,
            pltpu.SemaphoreType.DMA,
            pltpu.SemaphoreType.DMA,
        ],
    )(_sc_gather_body)

    out_call = pl.pallas_call(
        _main_out_kernel,
        grid=(grid,),
        in_specs=[
            pl.BlockSpec((bn, dim), lambda i: (i, 0)),
            pl.BlockSpec((dim, in_dim), lambda i: (0, 0)),
            pl.BlockSpec((in_dim,), lambda i: (0,)),
            pl.BlockSpec((heads, k, d), lambda i: (0, 0, 0)),
            pl.BlockSpec((heads, 1, k), lambda i: (0, 0, 0)),
            pl.BlockSpec((heads * k, dim), lambda i: (0, 0)),
        ],
        out_specs=[
            pl.BlockSpec((bn, dim), lambda i: (i, 0)),
            pl.BlockSpec((1, 1), lambda i: (0, 0)),
        ],
        out_shape=[
            jax.ShapeDtypeStruct((ns, dim), jnp.float32),
            jax.ShapeDtypeStruct((1, 1), jnp.float32),
        ],
    )

    outs = []
    loss = jnp.zeros((), jnp.float32)
    for s in range(_NSLICE):
        xs = lax.slice_in_dim(x, s * ns, (s + 1) * ns, axis=0)
        if s == _NSLICE - 1:
            out_s, loss_s = out_call(xs, W_in, b_in, embed, e2, P)
            outs.append(out_s)
            loss = loss + loss_s[0, 0]
            continue
        fidx_s, loss_s = main_call(xs, W_in, b_in, embed, e2)
        outs.append(sc_gather(P, fidx_s))
        loss = loss + loss_s[0, 0]

    out = jnp.concatenate(outs, axis=0)
    l_vq = loss / jnp.float32(heads * n * d)
    return (out, l_vq)
